# baseline (device time: 108986 ns/iter reference)
import numpy as np

import jax
import jax.numpy as jnp
from jax import lax
from jax.experimental import pallas as pl
from jax.experimental.pallas import tpu as pltpu

N_DEV = 32
CW_HOPS = N_DEV // 2
CCW_HOPS = N_DEV - 1 - CW_HOPS
N_SUB = 4


def _ring_order() -> np.ndarray:
    import distributed_mesh_v7x as dm

    try:
        mesh = dm.get_mesh("i", N_DEV)
        devs = list(mesh.devices)
        coords = [tuple(d.coords) for d in devs]
        to_logical = {c: i for i, c in enumerate(coords)}
        xs = sorted({c[0] for c in coords})
        ys = sorted({c[1] for c in coords})
        zs = sorted({c[2] for c in coords})
        if len(xs) != 2 or set(coords) != {
            (x, y, z) for x in xs for y in ys for z in zs
        }:
            return np.arange(N_DEV, dtype=np.int32)
        path_yz = []
        for zi, z in enumerate(zs):
            row = ys if zi % 2 == 0 else list(reversed(ys))
            path_yz.extend((y, z) for y in row)
        ring_coords = [(xs[0], y, z) for (y, z) in path_yz]
        ring_coords += [(xs[1], y, z) for (y, z) in reversed(path_yz)]
        return np.array([to_logical[c] for c in ring_coords], dtype=np.int32)
    except Exception:
        return np.arange(N_DEV, dtype=np.int32)


def kernel(x, w_mat, scale_x, scale_w):
    m_per, k = x.shape
    n_sh = w_mat.shape[1]

    x8 = x.astype(jnp.float8_e4m3fn)
    w8 = w_mat.astype(jnp.float8_e4m3fn)

    ring = _ring_order()
    pos_of = np.empty(N_DEV, dtype=np.int32)
    pos_of[ring] = np.arange(N_DEV, dtype=np.int32)
    ring_j = jnp.asarray(ring)

    my = lax.axis_index("i")
    r = jnp.take(jnp.asarray(pos_of), my)
    cw_chain = jnp.take(ring_j, jnp.mod(r - jnp.arange(CW_HOPS + 1), N_DEV))
    ccw_chain = jnp.take(ring_j, jnp.mod(r + jnp.arange(CCW_HOPS + 1), N_DEV))
    neighbors = jnp.stack([
        jnp.take(ring_j, jnp.mod(r - 1, N_DEV)),
        jnp.take(ring_j, jnp.mod(r + 1, N_DEV)),
    ]).astype(jnp.int32)

    def body(x_ref, w_ref, sx_ref, sw_ref, cw_ref, ccw_ref, nbr_ref,
             out_ref, gather_ref, w_bf16_ref,
             cw_send_sems, cw_recv_sems, ccw_send_sems, ccw_recv_sems):
        left = nbr_ref[0]
        right = nbr_ref[1]
        me = cw_ref[0]

        barrier_sem = pltpu.get_barrier_semaphore()
        pl.semaphore_signal(barrier_sem, inc=1, device_id=(left,),
                            device_id_type=pl.DeviceIdType.MESH)
        pl.semaphore_signal(barrier_sem, inc=1, device_id=(right,),
                            device_id_type=pl.DeviceIdType.MESH)
        pl.semaphore_wait(barrier_sem, 2)

        scale = sx_ref[0] * sw_ref[0]

        m_sub = m_per // N_SUB

        w_bf16_ref[...] = w_ref[...].astype(jnp.bfloat16)

        def gemm_store(origin, a):
            acc = lax.dot_general(
                a.astype(jnp.bfloat16), w_bf16_ref[...], (((1,), (0,)), ((), ())),
                preferred_element_type=jnp.float32)
            y = acc * scale
            out_ref[pl.ds(origin * m_per, m_per), :] = y * jax.nn.sigmoid(y)

        def descriptor(origin, sub, send_sem, recv_sem, target, src=None):
            sl = pl.ds(sub * m_sub, m_sub)
            return pltpu.make_async_remote_copy(
                src_ref=gather_ref.at[origin, sl] if src is None else src.at[sl],
                dst_ref=gather_ref.at[origin, sl],
                send_sem=send_sem,
                recv_sem=recv_sem,
                device_id=(target,),
                device_id_type=pl.DeviceIdType.MESH,
            )

        def send(origin, sub, send_sem, recv_sem, target, src=None):
            rdma = descriptor(origin, sub, send_sem, recv_sem, target, src)
            rdma.start()
            return rdma

        cw_rdmas = [
            send(me, s, cw_send_sems.at[0, s], cw_recv_sems.at[0, s], right,
                 src=x_ref)
            for s in range(N_SUB)
        ]
        ccw_rdmas = [
            send(me, s, ccw_send_sems.at[0, s], ccw_recv_sems.at[0, s], left,
                 src=x_ref)
            for s in range(N_SUB)
        ]
        gemm_store(me, x_ref[...])

        for t in range(CW_HOPS):
            cw_slot = cw_ref[t + 1]
            ccw_slot = ccw_ref[t + 1] if t < CCW_HOPS else None
            for s in range(N_SUB):
                cw = descriptor(cw_slot, s, cw_send_sems.at[t, s],
                                cw_recv_sems.at[t, s], right)
                cw.wait_recv()
                if t + 1 < CW_HOPS:
                    cw_rdmas.append(
                        send(cw_slot, s, cw_send_sems.at[t + 1, s],
                             cw_recv_sems.at[t + 1, s], right))
                if t < CCW_HOPS:
                    ccw = descriptor(ccw_slot, s, ccw_send_sems.at[t, s],
                                     ccw_recv_sems.at[t, s], left)
                    ccw.wait_recv()
                    if t + 1 < CCW_HOPS:
                        ccw_rdmas.append(
                            send(ccw_slot, s, ccw_send_sems.at[t + 1, s],
                                 ccw_recv_sems.at[t + 1, s], left))
            gemm_store(cw_slot, gather_ref[cw_slot])
            if t < CCW_HOPS:
                gemm_store(ccw_slot, gather_ref[ccw_slot])

        for rd in cw_rdmas:
            rd.wait_send()
        for rd in ccw_rdmas:
            rd.wait_send()

    return pl.pallas_call(
        body,
        out_shape=jax.ShapeDtypeStruct((N_DEV * m_per, n_sh), jnp.float32),
        in_specs=[
            pl.BlockSpec(memory_space=pltpu.VMEM),
            pl.BlockSpec(memory_space=pltpu.VMEM),
            pl.BlockSpec(memory_space=pltpu.SMEM),
            pl.BlockSpec(memory_space=pltpu.SMEM),
            pl.BlockSpec(memory_space=pltpu.SMEM),
            pl.BlockSpec(memory_space=pltpu.SMEM),
            pl.BlockSpec(memory_space=pltpu.SMEM),
        ],
        out_specs=pl.BlockSpec(memory_space=pltpu.VMEM),
        scratch_shapes=[
            pltpu.VMEM((N_DEV, m_per, k), jnp.float8_e4m3fn),
            pltpu.VMEM((k, n_sh), jnp.bfloat16),
            pltpu.SemaphoreType.DMA((CW_HOPS, N_SUB)),
            pltpu.SemaphoreType.DMA((CW_HOPS, N_SUB)),
            pltpu.SemaphoreType.DMA((CCW_HOPS, N_SUB)),
            pltpu.SemaphoreType.DMA((CCW_HOPS, N_SUB)),
        ],
        compiler_params=pltpu.CompilerParams(collective_id=0),
    )(x8, w8, scale_x, scale_w, cw_chain, ccw_chain, neighbors)


# device time: 81672 ns/iter; 1.3344x vs baseline; 1.3344x over previous
import numpy as np

import jax
import jax.numpy as jnp
from jax import lax
from jax.experimental import pallas as pl
from jax.experimental.pallas import tpu as pltpu

N_DEV = 32
N_POS = 16
CW_D = 8
CCW_D = 7
K_R = 1536
N_X = 17

_HAM16 = [(0, 0), (0, 1), (0, 2), (0, 3), (1, 3), (1, 2), (1, 1), (2, 1),
          (2, 2), (2, 3), (3, 3), (3, 2), (3, 1), (3, 0), (2, 0), (1, 0)]


def _ring_order() -> np.ndarray:
    import distributed_mesh_v7x as dm

    mesh = dm.get_mesh("i", N_DEV)
    coords = [tuple(d.coords) for d in mesh.devices]
    to_logical = {c: i for i, c in enumerate(coords)}
    xs = sorted({c[0] for c in coords})
    ys = sorted({c[1] for c in coords})
    zs = sorted({c[2] for c in coords})
    path_yz = []
    for zi, z in enumerate(zs):
        row = ys if zi % 2 == 0 else list(reversed(ys))
        path_yz.extend((y, z) for y in row)
    ring_coords = [(xs[0], y, z) for (y, z) in path_yz]
    ring_coords += [(xs[1], y, z) for (y, z) in reversed(path_yz)]
    return np.array([to_logical[c] for c in ring_coords], dtype=np.int32)


def _tables():
    import distributed_mesh_v7x as dm

    mesh = dm.get_mesh("i", N_DEV)
    coords = [tuple(d.coords) for d in mesh.devices]
    to_logical = {c: i for i, c in enumerate(coords)}
    xs = sorted({c[0] for c in coords})
    ys = sorted({c[1] for c in coords})
    zs = sorted({c[2] for c in coords})
    assert len(xs) == 2 and len(ys) == 4 and len(zs) == 4, (xs, ys, zs)

    id_at = np.zeros((2, N_POS), dtype=np.int32)
    for p, (yi, zi) in enumerate(_HAM16):
        for l in range(2):
            id_at[l, p] = to_logical[(xs[l], ys[yi], zs[zi])]

    cwf = np.zeros((N_DEV, CW_D + 1), dtype=np.int32)
    cwr = np.zeros((N_DEV, CW_D + 1), dtype=np.int32)
    ccwf = np.zeros((N_DEV, CCW_D + 1), dtype=np.int32)
    ccwr = np.zeros((N_DEV, CCW_D + 1), dtype=np.int32)
    nbr = np.zeros((N_DEV, 3), dtype=np.int32)
    for l in range(2):
        for r in range(N_POS):
            i = id_at[l, r]
            for d in range(CW_D + 1):
                cwf[i, d] = id_at[l, (r - d) % N_POS]
                cwr[i, d] = id_at[1 - l, (r - d) % N_POS]
            for d in range(CCW_D + 1):
                ccwf[i, d] = id_at[l, (r + d) % N_POS]
                ccwr[i, d] = id_at[1 - l, (r + d) % N_POS]
            nbr[i] = [id_at[l, (r - 1) % N_POS],
                      id_at[l, (r + 1) % N_POS],
                      id_at[1 - l, r]]
    return cwf, cwr, ccwf, ccwr, nbr


def kernel(x, w_mat, scale_x, scale_w):
    m_per, k = x.shape
    n_sh = w_mat.shape[1]
    k_x = k - K_R

    x8 = x.astype(jnp.float8_e4m3fn)
    w8 = w_mat.astype(jnp.float8_e4m3fn)

    cwf_t, cwr_t, ccwf_t, ccwr_t, nbr_t = _tables()
    my = lax.axis_index("i")
    take = lambda t: jnp.take(jnp.asarray(t), my, axis=0)
    cwf_j, cwr_j, ccwf_j, ccwr_j, nbr_j = map(
        take, (cwf_t, cwr_t, ccwf_t, ccwr_t, nbr_t))

    def body(x_ref, w_ref, sx_ref, sw_ref,
             cwf_ref, cwr_ref, ccwf_ref, ccwr_ref, nbr_ref,
             out_ref, gather_ref, w_bf16_ref,
             cwf_snd, cwf_rcv, cwr_snd, cwr_rcv,
             ccwf_snd, ccwf_rcv, ccwr_snd, ccwr_rcv,
             x_snd, x_rcv):
        left = nbr_ref[0]
        right = nbr_ref[1]
        partner = nbr_ref[2]
        me = cwf_ref[0]
        partner_chunk = cwr_ref[0]

        barrier_sem = pltpu.get_barrier_semaphore()
        for tgt in (left, right, partner):
            pl.semaphore_signal(barrier_sem, inc=1, device_id=(tgt,),
                                device_id_type=pl.DeviceIdType.MESH)
        pl.semaphore_wait(barrier_sem, 3)

        scale = sx_ref[0] * sw_ref[0]
        w_bf16_ref[...] = w_ref[...].astype(jnp.bfloat16)

        def gemm_store(origin, a):
            acc = lax.dot_general(
                a.astype(jnp.bfloat16), w_bf16_ref[...],
                (((1,), (0,)), ((), ())),
                preferred_element_type=jnp.float32)
            y = acc * scale
            out_ref[pl.ds(origin * m_per, m_per), :] = y * jax.nn.sigmoid(y)

        def full(cid):
            return gather_ref.at[cid]

        def rpart(cid):
            return gather_ref.at[cid, :, pl.ds(0, K_R)]

        def xpart(cid):
            return gather_ref.at[cid, :, pl.ds(K_R, k_x)]

        started = []

        def rcopy(src, dst, ssem, rsem, tgt, start):
            rdma = pltpu.make_async_remote_copy(
                src_ref=src, dst_ref=dst, send_sem=ssem, recv_sem=rsem,
                device_id=(tgt,), device_id_type=pl.DeviceIdType.MESH)
            if start:
                rdma.start()
                started.append(rdma)
            return rdma

        rcopy(x_ref, full(me), cwf_snd.at[1], cwf_rcv.at[1], right, True)
        rcopy(x_ref, full(me), ccwf_snd.at[1], ccwf_rcv.at[1], left, True)
        rcopy(x_ref.at[:, pl.ds(0, K_R)], rpart(me),
              x_snd.at[0], x_rcv.at[0], partner, True)
        rcopy(x_ref.at[:, pl.ds(K_R, k_x)], xpart(me),
              x_snd.at[1], x_rcv.at[1], partner, True)
        gemm_store(me, x_ref[...])

        rcopy(rpart(partner_chunk), rpart(partner_chunk),
              x_snd.at[0], x_rcv.at[0], partner, False).wait_recv()
        rcopy(rpart(partner_chunk), rpart(partner_chunk),
              cwr_snd.at[1], cwr_rcv.at[1], right, True)
        rcopy(rpart(partner_chunk), rpart(partner_chunk),
              ccwr_snd.at[1], ccwr_rcv.at[1], left, True)
        rcopy(xpart(partner_chunk), xpart(partner_chunk),
              x_snd.at[1], x_rcv.at[1], partner, False).wait_recv()
        gemm_store(partner_chunk, gather_ref[partner_chunk])

        for d in range(1, CW_D + 1):
            cf = cwf_ref[d]
            rcopy(full(cf), full(cf), cwf_snd.at[d], cwf_rcv.at[d],
                  right, False).wait_recv()
            if d < CW_D:
                rcopy(full(cf), full(cf), cwf_snd.at[d + 1],
                      cwf_rcv.at[d + 1], right, True)
            rcopy(xpart(cf), xpart(cf), x_snd.at[2 + (d - 1)],
                  x_rcv.at[2 + (d - 1)], partner, True)
            cr = cwr_ref[d]
            rcopy(rpart(cr), rpart(cr), cwr_snd.at[d], cwr_rcv.at[d],
                  right, False).wait_recv()
            if d < CW_D:
                rcopy(rpart(cr), rpart(cr), cwr_snd.at[d + 1],
                      cwr_rcv.at[d + 1], right, True)

            if d <= CCW_D:
                gf = ccwf_ref[d]
                rcopy(full(gf), full(gf), ccwf_snd.at[d], ccwf_rcv.at[d],
                      left, False).wait_recv()
                if d < CCW_D:
                    rcopy(full(gf), full(gf), ccwf_snd.at[d + 1],
                          ccwf_rcv.at[d + 1], left, True)
                rcopy(xpart(gf), xpart(gf), x_snd.at[10 + (d - 1)],
                      x_rcv.at[10 + (d - 1)], partner, True)
                gr = ccwr_ref[d]
                rcopy(rpart(gr), rpart(gr), ccwr_snd.at[d], ccwr_rcv.at[d],
                      left, False).wait_recv()
                if d < CCW_D:
                    rcopy(rpart(gr), rpart(gr), ccwr_snd.at[d + 1],
                          ccwr_rcv.at[d + 1], left, True)

            if d >= 2:
                pc = cwr_ref[d - 1]
                rcopy(xpart(pc), xpart(pc), x_snd.at[2 + (d - 2)],
                      x_rcv.at[2 + (d - 2)], partner, False).wait_recv()
                gemm_store(pc, gather_ref[pc])
                if d - 1 <= CCW_D:
                    qc = ccwr_ref[d - 1]
                    rcopy(xpart(qc), xpart(qc), x_snd.at[10 + (d - 2)],
                          x_rcv.at[10 + (d - 2)], partner, False).wait_recv()
                    gemm_store(qc, gather_ref[qc])

            gemm_store(cf, gather_ref[cf])
            if d <= CCW_D:
                gemm_store(gf, gather_ref[gf])

        pc = cwr_ref[CW_D]
        rcopy(xpart(pc), xpart(pc), x_snd.at[2 + (CW_D - 1)],
              x_rcv.at[2 + (CW_D - 1)], partner, False).wait_recv()
        gemm_store(pc, gather_ref[pc])

        for rdma in started:
            rdma.wait_send()

    return pl.pallas_call(
        body,
        out_shape=jax.ShapeDtypeStruct((N_DEV * m_per, n_sh), jnp.float32),
        in_specs=[
            pl.BlockSpec(memory_space=pltpu.VMEM),
            pl.BlockSpec(memory_space=pltpu.VMEM),
            pl.BlockSpec(memory_space=pltpu.SMEM),
            pl.BlockSpec(memory_space=pltpu.SMEM),
            pl.BlockSpec(memory_space=pltpu.SMEM),
            pl.BlockSpec(memory_space=pltpu.SMEM),
            pl.BlockSpec(memory_space=pltpu.SMEM),
            pl.BlockSpec(memory_space=pltpu.SMEM),
            pl.BlockSpec(memory_space=pltpu.SMEM),
        ],
        out_specs=pl.BlockSpec(memory_space=pltpu.VMEM),
        scratch_shapes=[
            pltpu.VMEM((N_DEV, m_per, k), jnp.float8_e4m3fn),
            pltpu.VMEM((k, n_sh), jnp.bfloat16),
            pltpu.SemaphoreType.DMA((CW_D + 1,)),
            pltpu.SemaphoreType.DMA((CW_D + 1,)),
            pltpu.SemaphoreType.DMA((CW_D + 1,)),
            pltpu.SemaphoreType.DMA((CW_D + 1,)),
            pltpu.SemaphoreType.DMA((CCW_D + 1,)),
            pltpu.SemaphoreType.DMA((CCW_D + 1,)),
            pltpu.SemaphoreType.DMA((CCW_D + 1,)),
            pltpu.SemaphoreType.DMA((CCW_D + 1,)),
            pltpu.SemaphoreType.DMA((N_X,)),
            pltpu.SemaphoreType.DMA((N_X,)),
        ],
        compiler_params=pltpu.CompilerParams(collective_id=0),
    )(x8, w8, scale_x, scale_w, cwf_j, cwr_j, ccwf_j, ccwr_j, nbr_j)
